# baseline (speedup 1.0000x reference)
import jax
import jax.numpy as jnp
from jax import lax
from jax.experimental import pallas as pl
from jax.experimental.pallas import tpu as pltpu
from jax.experimental.pallas import tpu_sc as plsc

B, S, D = 4, 8192, 768
NC, NS, L = 2, 16, 16
T = 6144
TCB = 3072
SEG = (S - T) // 8
K = 32
NBUF = 4
NJ = D // L
IDX_CAP = SEG + K


def _sc_body(feat_hbm, mask_hbm, out_hbm,
             mask_v, idx_v, gbuf, gbuf2, gbuf3, gbuf4, acc_v, r0row_v,
             part_v, shared_part, sem, sem2, sem3, sem4):
    c = lax.axis_index("c")
    s = lax.axis_index("s")
    b = c * 2 + s // 8
    base = b * S + T + (s % 8) * SEG

    pltpu.sync_copy(mask_hbm.at[pl.ds(base, SEG)], mask_v)

    r0v = jnp.full((L,), 0, dtype=jnp.int32) + base

    def fbody(t, _):
        idx_v[pl.ds(t * L, L)] = r0v
        return 0

    lax.fori_loop(0, IDX_CAP // L, fbody, 0)

    lane = lax.broadcasted_iota(jnp.int32, (L,), 0)

    def mbody(i, off):
        mv = mask_v[pl.ds(i * L, L)]
        mb = mv != 0.0
        mi = mb.astype(jnp.int32)
        pos = off + plsc.cumsum(mi) - 1
        plsc.store_scatter(idx_v, [pos], base + i * L + lane, mask=mb)
        return off + jnp.sum(mi)

    n = lax.fori_loop(0, SEG // L, mbody, jnp.int32(0))

    zero = jnp.zeros((L,), jnp.float32)

    def zbody(j, _):
        acc_v[pl.ds(j * L, L)] = zero
        return 0

    lax.fori_loop(0, NJ, zbody, 0)
    pltpu.sync_copy(feat_hbm.at[base], r0row_v)

    nrounds = (n + (K - 1)) // K

    bufs = (gbuf, gbuf2, gbuf3, gbuf4)
    sems = (sem, sem2, sem3, sem4)

    def start(r, buf, sm):
        @pl.when(r < nrounds)
        def _():
            pltpu.async_copy(feat_hbm.at[idx_v.at[pl.ds(r * K, K)]], buf, sm)

    def finish(r, buf, sm):
        @pl.when(r < nrounds)
        def _():
            pltpu.make_async_copy(feat_hbm.at[idx_v.at[pl.ds(r * K, K)]],
                                  buf, sm).wait()

            def jbody(j, _):
                dj = pl.ds(j * L, L)
                p0 = buf[0, dj]
                p1 = buf[1, dj]
                p2 = buf[2, dj]
                p3 = buf[3, dj]
                for k in range(4, K, 4):
                    p0 = p0 + buf[k, dj]
                    p1 = p1 + buf[k + 1, dj]
                    p2 = p2 + buf[k + 2, dj]
                    p3 = p3 + buf[k + 3, dj]
                plsc.addupdate(acc_v.at[dj], (p0 + p1) + (p2 + p3))
                return 0

            lax.fori_loop(0, NJ, jbody, 0)

    for t in range(NBUF):
        start(jnp.int32(t), bufs[t], sems[t])

    def pbody(r2, _):
        ra = NBUF * r2
        for t in range(NBUF):
            finish(ra + t, bufs[t], sems[t])
            start(ra + t + NBUF, bufs[t], sems[t])
        return 0

    lax.fori_loop(0, (nrounds + NBUF - 1) // NBUF, pbody, 0)

    pad = (nrounds * K - n).astype(jnp.float32)

    def cbody(j, _):
        dj = pl.ds(j * L, L)
        acc_v[dj] = acc_v[dj] - pad * r0row_v[dj]
        return 0

    lax.fori_loop(0, NJ, cbody, 0)

    pltpu.sync_copy(acc_v, shared_part.at[pl.ds(s * D, D)])
    plsc.subcore_barrier()

    @pl.when(s % 8 == 0)
    def _leader():
        pltpu.sync_copy(shared_part.at[pl.ds(s * D, 8 * D)], part_v)

        def lbody(j, _):
            tv = part_v[pl.ds(j * L, L)]
            for w in range(1, 8):
                tv = tv + part_v[pl.ds(w * D + j * L, L)]
            acc_v[pl.ds(j * L, L)] = tv
            return 0

        lax.fori_loop(0, NJ, lbody, 0)
        pltpu.sync_copy(acc_v, out_hbm.at[b, 0])


def _sc_sum(feat2d, maskf_flat):
    kern = pl.kernel(
        _sc_body,
        out_type=jax.ShapeDtypeStruct((B, 1, D), jnp.float32),
        mesh=plsc.VectorSubcoreMesh(core_axis_name="c", subcore_axis_name="s"),
        scratch_types=[
            pltpu.VMEM((SEG,), jnp.float32),
            pltpu.VMEM((IDX_CAP,), jnp.int32),
            pltpu.VMEM((K, D), jnp.float32),
            pltpu.VMEM((K, D), jnp.float32),
            pltpu.VMEM((K, D), jnp.float32),
            pltpu.VMEM((K, D), jnp.float32),
            pltpu.VMEM((D,), jnp.float32),
            pltpu.VMEM((D,), jnp.float32),
            pltpu.VMEM((8 * D,), jnp.float32),
            pltpu.VMEM_SHARED((NS * D,), jnp.float32),
            pltpu.SemaphoreType.DMA,
            pltpu.SemaphoreType.DMA,
            pltpu.SemaphoreType.DMA,
            pltpu.SemaphoreType.DMA,
        ],
        compiler_params=pltpu.CompilerParams(needs_layout_passes=False),
    )
    return kern(feat2d, maskf_flat)


def _tc_body(mask_ref, feat_ref, out_ref):
    i = pl.program_id(1)

    @pl.when(i == 0)
    def _():
        out_ref[0] = jnp.zeros((1, D), jnp.float32)

    out_ref[0] += lax.dot(mask_ref[0], feat_ref[0])


def _tc_sum(maskf3, features):
    return pl.pallas_call(
        _tc_body,
        grid=(B, T // TCB),
        in_specs=[
            pl.BlockSpec((1, 1, TCB), lambda b, i: (b * (T // TCB) + i, 0, 0)),
            pl.BlockSpec((1, TCB, D), lambda b, i: (b, i, 0)),
        ],
        out_specs=pl.BlockSpec((1, 1, D), lambda b, i: (b, 0, 0)),
        out_shape=jax.ShapeDtypeStruct((B, 1, D), jnp.float32),
    )(maskf3, features)


def _combine_body(tc_ref, sc_ref, mask_ref, out_ref):
    cnt = jnp.sum(mask_ref[...], axis=2, keepdims=True)
    denom = jnp.maximum(cnt, 1.0)
    out_ref[...] = (tc_ref[...] + sc_ref[...]) / denom


def _combine(tc_sum3d, sc_sum3d, maskf3d):
    return pl.pallas_call(
        _combine_body,
        out_shape=jax.ShapeDtypeStruct((B, 1, D), jnp.float32),
    )(tc_sum3d, sc_sum3d, maskf3d)


@jax.jit
def _pool(features, mask):
    feat2d = features.reshape(B * S, D)
    maskf = mask.astype(jnp.float32)
    maskf3 = maskf[:, :T].reshape(B * (T // TCB), 1, TCB)

    sc = _sc_sum(feat2d, maskf.reshape(-1))
    tc = _tc_sum(maskf3, features)
    return _combine(tc, sc, maskf.reshape(B, 1, S)).reshape(B, D)


def kernel(features, mask):
    return _pool(features, mask)


# speedup vs baseline: 1.0271x; 1.0271x over previous
import jax
import jax.numpy as jnp
from jax import lax
from jax.experimental import pallas as pl
from jax.experimental.pallas import tpu as pltpu
from jax.experimental.pallas import tpu_sc as plsc

B, S, D = 4, 8192, 768
NC, NS, L = 2, 16, 16
T = 4608
TCB = 1536
SEG = (S - T) // 8
K = 32
NBUF = 4
NJ = D // L
IDX_CAP = SEG + K


def _sc_body(feat_hbm, mask_hbm, out_hbm,
             mask_v, idx_v, gbuf, gbuf2, gbuf3, gbuf4, acc_v, r0row_v,
             part_v, shared_part, sem, sem2, sem3, sem4):
    c = lax.axis_index("c")
    s = lax.axis_index("s")
    b = c * 2 + s // 8
    base = b * S + T + (s % 8) * SEG

    pltpu.sync_copy(mask_hbm.at[pl.ds(base, SEG)], mask_v)

    r0v = jnp.full((L,), 0, dtype=jnp.int32) + base

    def fbody(t, _):
        idx_v[pl.ds(t * L, L)] = r0v
        return 0

    lax.fori_loop(0, IDX_CAP // L, fbody, 0)

    lane = lax.broadcasted_iota(jnp.int32, (L,), 0)

    def mbody(i, off):
        mv = mask_v[pl.ds(i * L, L)]
        mb = mv != 0.0
        mi = mb.astype(jnp.int32)
        pos = off + plsc.cumsum(mi) - 1
        plsc.store_scatter(idx_v, [pos], base + i * L + lane, mask=mb)
        return off + jnp.sum(mi)

    n = lax.fori_loop(0, SEG // L, mbody, jnp.int32(0))

    zero = jnp.zeros((L,), jnp.float32)

    def zbody(j, _):
        acc_v[pl.ds(j * L, L)] = zero
        return 0

    lax.fori_loop(0, NJ, zbody, 0)
    pltpu.sync_copy(feat_hbm.at[base], r0row_v)

    nrounds = (n + (K - 1)) // K

    bufs = (gbuf, gbuf2, gbuf3, gbuf4)
    sems = (sem, sem2, sem3, sem4)

    def start(r, buf, sm):
        @pl.when(r < nrounds)
        def _():
            pltpu.async_copy(feat_hbm.at[idx_v.at[pl.ds(r * K, K)]], buf, sm)

    def finish(r, buf, sm):
        @pl.when(r < nrounds)
        def _():
            pltpu.make_async_copy(feat_hbm.at[idx_v.at[pl.ds(r * K, K)]],
                                  buf, sm).wait()

            def jbody(j, _):
                dj = pl.ds(j * L, L)
                p0 = buf[0, dj]
                p1 = buf[1, dj]
                p2 = buf[2, dj]
                p3 = buf[3, dj]
                for k in range(4, K, 4):
                    p0 = p0 + buf[k, dj]
                    p1 = p1 + buf[k + 1, dj]
                    p2 = p2 + buf[k + 2, dj]
                    p3 = p3 + buf[k + 3, dj]
                plsc.addupdate(acc_v.at[dj], (p0 + p1) + (p2 + p3))
                return 0

            lax.fori_loop(0, NJ, jbody, 0)

    for t in range(NBUF):
        start(jnp.int32(t), bufs[t], sems[t])

    def pbody(r2, _):
        ra = NBUF * r2
        for t in range(NBUF):
            finish(ra + t, bufs[t], sems[t])
            start(ra + t + NBUF, bufs[t], sems[t])
        return 0

    lax.fori_loop(0, (nrounds + NBUF - 1) // NBUF, pbody, 0)

    pad = (nrounds * K - n).astype(jnp.float32)

    def cbody(j, _):
        dj = pl.ds(j * L, L)
        acc_v[dj] = acc_v[dj] - pad * r0row_v[dj]
        return 0

    lax.fori_loop(0, NJ, cbody, 0)

    pltpu.sync_copy(acc_v, shared_part.at[pl.ds(s * D, D)])
    plsc.subcore_barrier()

    @pl.when(s % 8 == 0)
    def _leader():
        pltpu.sync_copy(shared_part.at[pl.ds(s * D, 8 * D)], part_v)

        def lbody(j, _):
            tv = part_v[pl.ds(j * L, L)]
            for w in range(1, 8):
                tv = tv + part_v[pl.ds(w * D + j * L, L)]
            acc_v[pl.ds(j * L, L)] = tv
            return 0

        lax.fori_loop(0, NJ, lbody, 0)
        pltpu.sync_copy(acc_v, out_hbm.at[b, 0])


def _sc_sum(feat2d, maskf_flat):
    kern = pl.kernel(
        _sc_body,
        out_type=jax.ShapeDtypeStruct((B, 1, D), jnp.float32),
        mesh=plsc.VectorSubcoreMesh(core_axis_name="c", subcore_axis_name="s"),
        scratch_types=[
            pltpu.VMEM((SEG,), jnp.float32),
            pltpu.VMEM((IDX_CAP,), jnp.int32),
            pltpu.VMEM((K, D), jnp.float32),
            pltpu.VMEM((K, D), jnp.float32),
            pltpu.VMEM((K, D), jnp.float32),
            pltpu.VMEM((K, D), jnp.float32),
            pltpu.VMEM((D,), jnp.float32),
            pltpu.VMEM((D,), jnp.float32),
            pltpu.VMEM((8 * D,), jnp.float32),
            pltpu.VMEM_SHARED((NS * D,), jnp.float32),
            pltpu.SemaphoreType.DMA,
            pltpu.SemaphoreType.DMA,
            pltpu.SemaphoreType.DMA,
            pltpu.SemaphoreType.DMA,
        ],
        compiler_params=pltpu.CompilerParams(needs_layout_passes=False),
    )
    return kern(feat2d, maskf_flat)


def _tc_body(mask_ref, feat_ref, out_ref):
    i = pl.program_id(1)

    @pl.when(i == 0)
    def _():
        out_ref[0] = jnp.zeros((1, D), jnp.float32)

    out_ref[0] += lax.dot(mask_ref[0], feat_ref[0])


def _tc_sum(maskf3, features):
    return pl.pallas_call(
        _tc_body,
        grid=(B, T // TCB),
        in_specs=[
            pl.BlockSpec((1, 1, TCB), lambda b, i: (b * (T // TCB) + i, 0, 0)),
            pl.BlockSpec((1, TCB, D), lambda b, i: (b, i, 0)),
        ],
        out_specs=pl.BlockSpec((1, 1, D), lambda b, i: (b, 0, 0)),
        out_shape=jax.ShapeDtypeStruct((B, 1, D), jnp.float32),
    )(maskf3, features)


def _combine_body(tc_ref, sc_ref, mask_ref, out_ref):
    cnt = jnp.sum(mask_ref[...], axis=2, keepdims=True)
    denom = jnp.maximum(cnt, 1.0)
    out_ref[...] = (tc_ref[...] + sc_ref[...]) / denom


def _combine(tc_sum3d, sc_sum3d, maskf3d):
    return pl.pallas_call(
        _combine_body,
        out_shape=jax.ShapeDtypeStruct((B, 1, D), jnp.float32),
    )(tc_sum3d, sc_sum3d, maskf3d)


@jax.jit
def _pool(features, mask):
    feat2d = features.reshape(B * S, D)
    maskf = mask.astype(jnp.float32)
    maskf3 = maskf[:, :T].reshape(B * (T // TCB), 1, TCB)

    sc = _sc_sum(feat2d, maskf.reshape(-1))
    tc = _tc_sum(maskf3, features)
    return _combine(tc, sc, maskf.reshape(B, 1, S)).reshape(B, D)


def kernel(features, mask):
    return _pool(features, mask)


# speedup vs baseline: 1.0436x; 1.0161x over previous
import jax
import jax.numpy as jnp
from jax import lax
from jax.experimental import pallas as pl
from jax.experimental.pallas import tpu as pltpu
from jax.experimental.pallas import tpu_sc as plsc

B, S, D = 4, 8192, 768
NC, NS, L = 2, 16, 16
T = 5120
TCB = 2560
SEG = (S - T) // 8
K = 32
NBUF = 4
NJ = D // L
IDX_CAP = SEG + K


def _sc_body(feat_hbm, mask_hbm, out_hbm,
             mask_v, idx_v, gbuf, gbuf2, gbuf3, gbuf4, acc_v, r0row_v,
             part_v, shared_part, sem, sem2, sem3, sem4):
    c = lax.axis_index("c")
    s = lax.axis_index("s")
    b = c * 2 + s // 8
    base = b * S + T + (s % 8) * SEG

    pltpu.sync_copy(mask_hbm.at[pl.ds(base, SEG)], mask_v)

    r0v = jnp.full((L,), 0, dtype=jnp.int32) + base

    def fbody(t, _):
        idx_v[pl.ds(t * L, L)] = r0v
        return 0

    lax.fori_loop(0, IDX_CAP // L, fbody, 0)

    lane = lax.broadcasted_iota(jnp.int32, (L,), 0)

    def mbody(i, off):
        mv = mask_v[pl.ds(i * L, L)]
        mb = mv != 0.0
        mi = mb.astype(jnp.int32)
        pos = off + plsc.cumsum(mi) - 1
        plsc.store_scatter(idx_v, [pos], base + i * L + lane, mask=mb)
        return off + jnp.sum(mi)

    n = lax.fori_loop(0, SEG // L, mbody, jnp.int32(0))

    zero = jnp.zeros((L,), jnp.float32)

    def zbody(j, _):
        acc_v[pl.ds(j * L, L)] = zero
        return 0

    lax.fori_loop(0, NJ, zbody, 0)
    pltpu.sync_copy(feat_hbm.at[base], r0row_v)

    nrounds = (n + (K - 1)) // K

    bufs = (gbuf, gbuf2, gbuf3, gbuf4)
    sems = (sem, sem2, sem3, sem4)

    def start(r, buf, sm):
        @pl.when(r < nrounds)
        def _():
            pltpu.async_copy(feat_hbm.at[idx_v.at[pl.ds(r * K, K)]], buf, sm)

    def finish(r, buf, sm):
        @pl.when(r < nrounds)
        def _():
            pltpu.make_async_copy(feat_hbm.at[idx_v.at[pl.ds(r * K, K)]],
                                  buf, sm).wait()

            def jbody(j, _):
                dj = pl.ds(j * L, L)
                p0 = buf[0, dj]
                p1 = buf[1, dj]
                p2 = buf[2, dj]
                p3 = buf[3, dj]
                for k in range(4, K, 4):
                    p0 = p0 + buf[k, dj]
                    p1 = p1 + buf[k + 1, dj]
                    p2 = p2 + buf[k + 2, dj]
                    p3 = p3 + buf[k + 3, dj]
                plsc.addupdate(acc_v.at[dj], (p0 + p1) + (p2 + p3))
                return 0

            lax.fori_loop(0, NJ, jbody, 0)

    for t in range(NBUF):
        start(jnp.int32(t), bufs[t], sems[t])

    def pbody(r2, _):
        ra = NBUF * r2
        for t in range(NBUF):
            finish(ra + t, bufs[t], sems[t])
            start(ra + t + NBUF, bufs[t], sems[t])
        return 0

    lax.fori_loop(0, (nrounds + NBUF - 1) // NBUF, pbody, 0)

    pad = (nrounds * K - n).astype(jnp.float32)

    def cbody(j, _):
        dj = pl.ds(j * L, L)
        acc_v[dj] = acc_v[dj] - pad * r0row_v[dj]
        return 0

    lax.fori_loop(0, NJ, cbody, 0)

    pltpu.sync_copy(acc_v, shared_part.at[pl.ds(s * D, D)])
    plsc.subcore_barrier()

    @pl.when(s % 8 == 0)
    def _leader():
        pltpu.sync_copy(shared_part.at[pl.ds(s * D, 8 * D)], part_v)

        def lbody(j, _):
            tv = part_v[pl.ds(j * L, L)]
            for w in range(1, 8):
                tv = tv + part_v[pl.ds(w * D + j * L, L)]
            acc_v[pl.ds(j * L, L)] = tv
            return 0

        lax.fori_loop(0, NJ, lbody, 0)
        pltpu.sync_copy(acc_v, out_hbm.at[b, 0])


def _sc_sum(feat2d, maskf_flat):
    kern = pl.kernel(
        _sc_body,
        out_type=jax.ShapeDtypeStruct((B, 1, D), jnp.float32),
        mesh=plsc.VectorSubcoreMesh(core_axis_name="c", subcore_axis_name="s"),
        scratch_types=[
            pltpu.VMEM((SEG,), jnp.float32),
            pltpu.VMEM((IDX_CAP,), jnp.int32),
            pltpu.VMEM((K, D), jnp.float32),
            pltpu.VMEM((K, D), jnp.float32),
            pltpu.VMEM((K, D), jnp.float32),
            pltpu.VMEM((K, D), jnp.float32),
            pltpu.VMEM((D,), jnp.float32),
            pltpu.VMEM((D,), jnp.float32),
            pltpu.VMEM((8 * D,), jnp.float32),
            pltpu.VMEM_SHARED((NS * D,), jnp.float32),
            pltpu.SemaphoreType.DMA,
            pltpu.SemaphoreType.DMA,
            pltpu.SemaphoreType.DMA,
            pltpu.SemaphoreType.DMA,
        ],
        compiler_params=pltpu.CompilerParams(needs_layout_passes=False),
    )
    return kern(feat2d, maskf_flat)


def _tc_body(mask_ref, feat_ref, out_ref):
    i = pl.program_id(1)

    @pl.when(i == 0)
    def _():
        out_ref[0] = jnp.zeros((1, D), jnp.float32)

    out_ref[0] += lax.dot(mask_ref[0], feat_ref[0])


def _tc_sum(maskf3, features):
    return pl.pallas_call(
        _tc_body,
        grid=(B, T // TCB),
        in_specs=[
            pl.BlockSpec((1, 1, TCB), lambda b, i: (b * (T // TCB) + i, 0, 0)),
            pl.BlockSpec((1, TCB, D), lambda b, i: (b, i, 0)),
        ],
        out_specs=pl.BlockSpec((1, 1, D), lambda b, i: (b, 0, 0)),
        out_shape=jax.ShapeDtypeStruct((B, 1, D), jnp.float32),
    )(maskf3, features)


def _combine_body(tc_ref, sc_ref, mask_ref, out_ref):
    cnt = jnp.sum(mask_ref[...], axis=2, keepdims=True)
    denom = jnp.maximum(cnt, 1.0)
    out_ref[...] = (tc_ref[...] + sc_ref[...]) / denom


def _combine(tc_sum3d, sc_sum3d, maskf3d):
    return pl.pallas_call(
        _combine_body,
        out_shape=jax.ShapeDtypeStruct((B, 1, D), jnp.float32),
    )(tc_sum3d, sc_sum3d, maskf3d)


@jax.jit
def _pool(features, mask):
    feat2d = features.reshape(B * S, D)
    maskf = mask.astype(jnp.float32)
    maskf3 = maskf[:, :T].reshape(B * (T // TCB), 1, TCB)

    sc = _sc_sum(feat2d, maskf.reshape(-1))
    tc = _tc_sum(maskf3, features)
    return _combine(tc, sc, maskf.reshape(B, 1, S)).reshape(B, D)


def kernel(features, mask):
    return _pool(features, mask)
